# baseline (speedup 1.0000x reference)
import jax
import jax.numpy as jnp
from jax.experimental import pallas as pl
from jax.experimental.pallas import tpu as pltpu

_BK = 2048


def _tanh_block(x_ref, o_ref):
    o_ref[...] = jnp.tanh(x_ref[...])


def kernel(mean, stddev):
    del stddev
    m, n = mean.shape
    grid = (pl.cdiv(n, _BK),)
    return pl.pallas_call(
        _tanh_block,
        grid=grid,
        in_specs=[pl.BlockSpec((m, _BK), lambda j: (0, j))],
        out_specs=pl.BlockSpec((m, _BK), lambda j: (0, j)),
        out_shape=jax.ShapeDtypeStruct((m, n), jnp.float32),
    )(mean)
